# Initial kernel scaffold; baseline (speedup 1.0000x reference)
import jax
import jax.numpy as jnp
from jax.experimental import pallas as pl


def kernel(feat, edge_index, labels, W_transh, b_transh, W_lin, b_lin, W_lin1, b_lin1):
    raise NotImplementedError("write your pallas kernel here")


# speedup vs baseline: 5.2509x; 5.2509x over previous
import functools

import jax
import jax.numpy as jnp
from jax import lax
from jax.experimental import pallas as pl
from jax.experimental.pallas import tpu as pltpu
from jax.experimental.pallas import tpu_sc as plsc

N = 10000
E = 320000
D = 128
OUT = 128
THETAS = [[1.0, -0.5, 0.25], [0.5, 0.5, -0.25], [0.25, -0.25, 0.5], [0.1, 0.2, 0.3]]
NEG_SLOPE = 0.01

NC = 2
NSUB = 16
NW = NC * NSUB
SEG = 10240
S2 = 2 * SEG

_MESH = plsc.VectorSubcoreMesh(core_axis_name="c", subcore_axis_name="s")


def _sc_prep(src, dst, labels, zeros_deg):
    EPT = E // NW
    B = 80
    NCH = EPT // B
    DSL = S2 // NSUB

    @functools.partial(
        pl.kernel,
        mesh=_MESH,
        compiler_params=pltpu.CompilerParams(needs_layout_passes=False,
                                             use_tc_tiling_on_sc=False),
        out_type=(
            jax.ShapeDtypeStruct((NC * E,), jnp.int32),
            jax.ShapeDtypeStruct((NC * E,), jnp.int32),
            jax.ShapeDtypeStruct((E,), jnp.int32),
            jax.ShapeDtypeStruct((NC * S2,), jnp.float32),
        ),
        scratch_types=[
            pltpu.VMEM((N,), jnp.int32),
            pltpu.VMEM((B,), jnp.int32),
            pltpu.VMEM((B,), jnp.int32),
            pltpu.VMEM((B,), jnp.int32),
            pltpu.VMEM((B,), jnp.int32),
            pltpu.VMEM((B,), jnp.int32),
            pltpu.VMEM((B,), jnp.int32),
            pltpu.VMEM((B,), jnp.int32),
            pltpu.VMEM((B,), jnp.float32),
            pltpu.VMEM_SHARED((S2,), jnp.float32),
        ],
    )
    def k(src_h, dst_h, lab_h, zdeg_h, srcx_h, srcx0_h, dst2_h, degs_h,
          lab_v, src_v, dst_v, sx0_v, sx1_v, s00_v, s01_v, d2_v, ones_v, dacc):
        c = lax.axis_index("c")
        s = lax.axis_index("s")
        wid = c * NSUB + s
        pltpu.sync_copy(lab_h, lab_v)
        pltpu.sync_copy(zdeg_h.at[pl.ds(s * DSL, DSL)], dacc.at[pl.ds(s * DSL, DSL)])
        for j in range(B // 16):
            ones_v[pl.ds(16 * j, 16)] = jnp.full((16,), 1.0, jnp.float32)
        plsc.subcore_barrier()

        def body(g, _):
            base = wid * EPT + g * B
            pltpu.sync_copy(src_h.at[pl.ds(base, B)], src_v)
            pltpu.sync_copy(dst_h.at[pl.ds(base, B)], dst_v)
            for j in range(B // 16):
                sl = pl.ds(16 * j, 16)
                sv = src_v[sl]
                dv = dst_v[sl]
                ls = plsc.load_gather(lab_v, [sv])
                ld = plsc.load_gather(lab_v, [dv])
                isneg = ls != ld
                s2 = sv + jnp.where(isneg, jnp.int32(N), jnp.int32(0))
                sx0_v[sl] = 2 * s2
                sx1_v[sl] = 2 * s2 + 1
                s00_v[sl] = 2 * sv
                s01_v[sl] = 2 * sv + 1
                d2_v[sl] = dv + jnp.where(isneg, jnp.int32(SEG), jnp.int32(0))
            pltpu.sync_copy(sx0_v, srcx_h.at[pl.ds(base, B)])
            pltpu.sync_copy(sx1_v, srcx_h.at[pl.ds(E + base, B)])
            pltpu.sync_copy(s00_v, srcx0_h.at[pl.ds(base, B)])
            pltpu.sync_copy(s01_v, srcx0_h.at[pl.ds(E + base, B)])
            pltpu.sync_copy(d2_v, dst2_h.at[pl.ds(base, B)])
            pltpu.sync_copy(ones_v, dacc.at[d2_v], add=True)
            return ()

        lax.fori_loop(0, NCH, body, (), unroll=False)
        plsc.subcore_barrier()
        pltpu.sync_copy(dacc.at[pl.ds(s * DSL, DSL)],
                        degs_h.at[pl.ds(c * S2 + s * DSL, DSL)])

    return k(src, dst, labels, zeros_deg)


def _sc_seg_pass(table_v, srcx, dst2, zeros, S):
    EPS = E // NSUB
    B = 80
    NCH = EPS // B
    RPT = S // NSUB

    @functools.partial(
        pl.kernel,
        mesh=_MESH,
        compiler_params=pltpu.CompilerParams(use_tc_tiling_on_sc=False),
        out_type=jax.ShapeDtypeStruct((NC, S, 64), jnp.float32),
        scratch_types=[
            pltpu.VMEM((B,), jnp.int32),
            pltpu.VMEM((B,), jnp.int32),
            pltpu.VMEM((B, 64), jnp.float32),
            pltpu.VMEM_SHARED((S, 64), jnp.float32),
            pltpu.SemaphoreType.DMA,
        ],
    )
    def k(table_h, srcx_h, dst_h, zeros_h, out_h, idx_v, didx_v, rows_v, acc, sem):
        c = lax.axis_index("c")
        s = lax.axis_index("s")
        pltpu.sync_copy(zeros_h.at[pl.ds(s * RPT, RPT)], acc.at[pl.ds(s * RPT, RPT)])
        plsc.subcore_barrier()

        def body(g, _):
            base = s * EPS + g * B
            pltpu.sync_copy(srcx_h.at[pl.ds(c * E + base, B)], idx_v)
            pltpu.sync_copy(dst_h.at[pl.ds(base, B)], didx_v)
            pltpu.async_copy(table_h.at[idx_v], rows_v, sem).wait()
            pltpu.sync_copy(rows_v, acc.at[didx_v], add=True)
            return ()

        lax.fori_loop(0, NCH, body, (), unroll=False)
        plsc.subcore_barrier()
        pltpu.sync_copy(acc.at[pl.ds(s * RPT, RPT)],
                        out_h.at[c, pl.ds(s * RPT, RPT)])

    return k(table_v, srcx, dst2, zeros)


RB = 1000


def _cat_halves(a2):
    return jnp.concatenate([a2[0], a2[1]], axis=-1)


def _tc_init(feat, degs, Wt, bt, C1s):
    def body(f_r, dg_r, wt_r, bt_r, c1_r, g1_r, th_r, d2_r, dsq_r, dv_r, y0_r):
        f = f_r[...]
        pd = dg_r[0, 0] + dg_r[1, 0]
        nd = dg_r[0, 1] + dg_r[1, 1]
        pdc = jnp.maximum(pd, 1.0)
        ndc = jnp.maximum(nd, 1.0)
        adc = jnp.maximum(pd + nd, 1.0)
        dp = lax.rsqrt(pdc)
        dn = lax.rsqrt(ndc)
        g1_r[0] = f * dp
        g1_r[1] = f * dn
        th_r[...] = jnp.dot(f, wt_r[...], preferred_element_type=jnp.float32) + bt_r[...]
        d2_r[0] = dp * dp
        d2_r[1] = dn * dn
        dsq_r[0] = jnp.sqrt(pdc)
        dsq_r[1] = jnp.sqrt(ndc)
        dv_r[...] = lax.rsqrt(adc)
        y0_r[...] = jnp.dot(f, c1_r[...], preferred_element_type=jnp.float32)

    grid = (N // RB,)
    return pl.pallas_call(
        body,
        grid=grid,
        in_specs=[
            pl.BlockSpec((RB, D), lambda i: (i, 0)),
            pl.BlockSpec((NC, 2, RB, 1), lambda i: (0, 0, i, 0)),
            pl.BlockSpec((D, OUT), lambda i: (0, 0)),
            pl.BlockSpec((1, OUT), lambda i: (0, 0)),
            pl.BlockSpec((D, OUT), lambda i: (0, 0)),
        ],
        out_specs=[
            pl.BlockSpec((2, RB, D), lambda i: (0, i, 0)),
            pl.BlockSpec((RB, OUT), lambda i: (i, 0)),
            pl.BlockSpec((2, RB, 1), lambda i: (0, i, 0)),
            pl.BlockSpec((2, RB, 1), lambda i: (0, i, 0)),
            pl.BlockSpec((RB, 1), lambda i: (i, 0)),
            pl.BlockSpec((RB, OUT), lambda i: (i, 0)),
        ],
        out_shape=[
            jax.ShapeDtypeStruct((2, N, D), jnp.float32),
            jax.ShapeDtypeStruct((N, OUT), jnp.float32),
            jax.ShapeDtypeStruct((2, N, 1), jnp.float32),
            jax.ShapeDtypeStruct((2, N, 1), jnp.float32),
            jax.ShapeDtypeStruct((N, 1), jnp.float32),
            jax.ShapeDtypeStruct((N, OUT), jnp.float32),
        ],
    )(feat, degs, Wt, bt, C1s)


def _tc_update(G, AGGc, D2, Dsq, Ck, Y):
    def body(g_r, a_r, d2_r, dsq_r, ck_r, y_r, gn_r, yn_r):
        agg = jnp.stack([_cat_halves(a_r[:, 0]), _cat_halves(a_r[:, 1])])
        g = g_r[...] - d2_r[...] * agg
        gn_r[...] = g
        f = g * dsq_r[...]
        yn_r[...] = (y_r[...]
                     + jnp.dot(f[0], ck_r[0], preferred_element_type=jnp.float32)
                     + jnp.dot(f[1], ck_r[1], preferred_element_type=jnp.float32))

    grid = (N // RB,)
    return pl.pallas_call(
        body,
        grid=grid,
        in_specs=[
            pl.BlockSpec((2, RB, D), lambda i: (0, i, 0)),
            pl.BlockSpec((NC, 2, RB, 64), lambda i: (0, 0, i, 0)),
            pl.BlockSpec((2, RB, 1), lambda i: (0, i, 0)),
            pl.BlockSpec((2, RB, 1), lambda i: (0, i, 0)),
            pl.BlockSpec((2, D, OUT), lambda i: (0, 0, 0)),
            pl.BlockSpec((RB, OUT), lambda i: (i, 0)),
        ],
        out_specs=[
            pl.BlockSpec((2, RB, D), lambda i: (0, i, 0)),
            pl.BlockSpec((RB, OUT), lambda i: (i, 0)),
        ],
        out_shape=[
            jax.ShapeDtypeStruct((2, N, D), jnp.float32),
            jax.ShapeDtypeStruct((N, OUT), jnp.float32),
        ],
    )(G, AGGc, D2, Dsq, Ck, Y)


def _tc_final(G4, AGG4c, D2, Dsq, C5, Y, bl1, feat, A0c, Dv, Wa, Wb, bl):
    def body(g_r, a_r, d2_r, dsq_r, c5_r, y_r, bl1_r, f_r, a0_r, dv_r,
             wa_r, wb_r, bl_r, hso_r, hspn_r):
        agg = jnp.stack([_cat_halves(a_r[:, 0]), _cat_halves(a_r[:, 1])])
        g = g_r[...] - d2_r[...] * agg
        f5 = g * dsq_r[...]
        y = (y_r[...]
             + jnp.dot(f5[0], c5_r[0], preferred_element_type=jnp.float32)
             + jnp.dot(f5[1], c5_r[1], preferred_element_type=jnp.float32)
             + bl1_r[...])
        hspn_r[...] = jnp.where(y >= 0, y, NEG_SLOPE * y)
        cvec = _cat_halves(a0_r[...]) * dv_r[...]
        z = (jnp.dot(f_r[...], wa_r[...], preferred_element_type=jnp.float32)
             + jnp.dot(cvec, wb_r[...], preferred_element_type=jnp.float32)
             + bl_r[...])
        hso_r[...] = jnp.where(z >= 0, z, NEG_SLOPE * z)

    grid = (N // RB,)
    return pl.pallas_call(
        body,
        grid=grid,
        in_specs=[
            pl.BlockSpec((2, RB, D), lambda i: (0, i, 0)),
            pl.BlockSpec((NC, 2, RB, 64), lambda i: (0, 0, i, 0)),
            pl.BlockSpec((2, RB, 1), lambda i: (0, i, 0)),
            pl.BlockSpec((2, RB, 1), lambda i: (0, i, 0)),
            pl.BlockSpec((2, D, OUT), lambda i: (0, 0, 0)),
            pl.BlockSpec((RB, OUT), lambda i: (i, 0)),
            pl.BlockSpec((1, OUT), lambda i: (0, 0)),
            pl.BlockSpec((RB, D), lambda i: (i, 0)),
            pl.BlockSpec((NC, RB, 64), lambda i: (0, i, 0)),
            pl.BlockSpec((RB, 1), lambda i: (i, 0)),
            pl.BlockSpec((D, OUT), lambda i: (0, 0)),
            pl.BlockSpec((D, OUT), lambda i: (0, 0)),
            pl.BlockSpec((1, OUT), lambda i: (0, 0)),
        ],
        out_specs=[
            pl.BlockSpec((RB, OUT), lambda i: (i, 0)),
            pl.BlockSpec((RB, OUT), lambda i: (i, 0)),
        ],
        out_shape=[
            jax.ShapeDtypeStruct((N, OUT), jnp.float32),
            jax.ShapeDtypeStruct((N, OUT), jnp.float32),
        ],
    )(G4, AGG4c, D2, Dsq, C5, Y, bl1, feat, A0c, Dv, Wa, Wb, bl)


def kernel(feat, edge_index, labels, W_transh, b_transh, W_lin, b_lin,
           W_lin1, b_lin1):
    src = edge_index[0]
    dst = edge_index[1]

    t0, t1, t2, t3 = THETAS
    a = [sum(t) for t in THETAS]
    b = [-(2 * i * a[i] + THETAS[i][1] + 2 * THETAS[i][2]) for i in range(4)]
    WL = [W_lin[D * i:D * (i + 1)] for i in range(4)]
    Wa = a[0] * WL[0] + a[1] * WL[1] + a[2] * WL[2] + a[3] * WL[3]
    Wb = b[0] * WL[0] + b[1] * WL[1] + b[2] * WL[2] + b[3] * WL[3]
    V = [W_lin1[D * i:D * (i + 1)] for i in range(4)]
    A = [t0[0] * V[0], t0[1] * V[0], t0[2] * V[0] + t1[0] * V[1],
         t1[1] * V[1], t1[2] * V[1]]
    Bm = [t2[0] * V[2], t2[1] * V[2], t2[2] * V[2] + t3[0] * V[3],
          t3[1] * V[3], t3[2] * V[3]]
    C = [jnp.stack([Ak, Bk]) for Ak, Bk in zip(A, Bm)]
    C1s = A[0] + Bm[0]

    zeros_deg = jnp.zeros((S2,), jnp.float32)
    zeros_n = jnp.zeros((SEG, 64), jnp.float32)
    zeros_2n = jnp.zeros((S2, 64), jnp.float32)

    srcx, srcx0, dst2, degs_flat = _sc_prep(src, dst, labels, zeros_deg)
    degs = degs_flat.reshape(NC, 2, SEG, 1)

    feat_v = feat.reshape(2 * N, 64)
    agg0c = _sc_seg_pass(feat_v, srcx0, dst, zeros_n, SEG)

    bt = b_transh.reshape(1, OUT)
    bl = b_lin.reshape(1, OUT)
    bl1 = b_lin1.reshape(1, OUT)
    G, transh, D2, Dsq, Dv, Y = _tc_init(feat, degs, W_transh, bt, C1s)

    for k in range(1, 4):
        AGGc = _sc_seg_pass(G.reshape(4 * N, 64), srcx, dst2, zeros_2n, S2)
        AGGc = AGGc.reshape(NC, 2, SEG, 64)
        G, Y = _tc_update(G, AGGc, D2, Dsq, C[k], Y)
    AGG4c = _sc_seg_pass(G.reshape(4 * N, 64), srcx, dst2, zeros_2n, S2)
    AGG4c = AGG4c.reshape(NC, 2, SEG, 64)

    hs_o_out, hs_pn_out = _tc_final(G, AGG4c, D2, Dsq, C[4], Y, bl1,
                                    feat, agg0c, Dv, Wa, Wb, bl)
    return (hs_o_out, hs_pn_out, transh)


# speedup vs baseline: 6.5988x; 1.2567x over previous
import functools

import jax
import jax.numpy as jnp
from jax import lax
from jax.experimental import pallas as pl
from jax.experimental.pallas import tpu as pltpu
from jax.experimental.pallas import tpu_sc as plsc

N = 10000
E = 320000
D = 128
OUT = 128
THETAS = [[1.0, -0.5, 0.25], [0.5, 0.5, -0.25], [0.25, -0.25, 0.5], [0.1, 0.2, 0.3]]
NEG_SLOPE = 0.01

NC = 2
NSUB = 16
NW = NC * NSUB
SEG = 10240
S2 = 2 * SEG
EPS = E // NSUB
B = 128
EPP = 20480
EPAD = NSUB * EPP
NCH = EPP // B
PAD_D2 = S2 - B
PAD_D0 = SEG - 8

_MESH = plsc.VectorSubcoreMesh(core_axis_name="c", subcore_axis_name="s")


def _sc_prep(src, dst, labels, zeros_deg):
    EPT = E // NW
    BP = 80
    NCP = EPT // BP
    NPAD = EPP - 2 * EPT
    DSL = S2 // NSUB

    @functools.partial(
        pl.kernel,
        mesh=_MESH,
        compiler_params=pltpu.CompilerParams(needs_layout_passes=False,
                                             use_tc_tiling_on_sc=False),
        out_type=(
            jax.ShapeDtypeStruct((NC * EPAD,), jnp.int32),
            jax.ShapeDtypeStruct((NC * EPAD,), jnp.int32),
            jax.ShapeDtypeStruct((EPAD,), jnp.int32),
            jax.ShapeDtypeStruct((EPAD,), jnp.int32),
            jax.ShapeDtypeStruct((NC * S2,), jnp.float32),
        ),
        scratch_types=[
            pltpu.VMEM((N,), jnp.int32),
            pltpu.VMEM((EPT,), jnp.int32),
            pltpu.VMEM((EPT,), jnp.int32),
            pltpu.VMEM((EPT,), jnp.int32),
            pltpu.VMEM((EPT,), jnp.int32),
            pltpu.VMEM((EPT,), jnp.int32),
            pltpu.VMEM((EPT,), jnp.int32),
            pltpu.VMEM((NCP, BP), jnp.int32),
            pltpu.VMEM((BP,), jnp.float32),
            pltpu.VMEM((NPAD,), jnp.int32),
            pltpu.VMEM((NPAD,), jnp.int32),
            pltpu.VMEM((NPAD,), jnp.int32),
            pltpu.VMEM_SHARED((S2,), jnp.float32),
            pltpu.SemaphoreType.DMA,
            pltpu.SemaphoreType.DMA,
        ],
    )
    def k(src_h, dst_h, lab_h, zdeg_h, srcx_h, srcx0_h, dst2_h, dst0_h, degs_h,
          lab_v, src_v, dst_v, sx0_v, sx1_v, s00_v, s01_v, d2_v,
          ones_v, pz_v, pd2_v, pd0_v, dacc, sd, sd2):
        c = lax.axis_index("c")
        s = lax.axis_index("s")
        wid = c * NSUB + s
        q = wid // 2
        r = wid % 2
        ebase = wid * EPT
        obase = q * EPP + r * EPT
        pltpu.sync_copy(lab_h, lab_v)
        pltpu.sync_copy(src_h.at[pl.ds(ebase, EPT)], src_v)
        pltpu.sync_copy(dst_h.at[pl.ds(ebase, EPT)], dst_v)
        pltpu.sync_copy(zdeg_h.at[pl.ds(s * DSL, DSL)], dacc.at[pl.ds(s * DSL, DSL)])
        for j in range(BP // 16):
            ones_v[pl.ds(16 * j, 16)] = jnp.full((16,), 1.0, jnp.float32)
        for j in range(NPAD // 16):
            sl = pl.ds(16 * j, 16)
            pz_v[sl] = jnp.zeros((16,), jnp.int32)
            pd2_v[sl] = jnp.full((16,), PAD_D2, jnp.int32)
            pd0_v[sl] = jnp.full((16,), PAD_D0, jnp.int32)

        def cbody(g, _):
            for j in range(BP // 16):
                sl = pl.ds(g * BP + 16 * j, 16)
                sv = src_v[sl]
                dv = dst_v[sl]
                ls = plsc.load_gather(lab_v, [sv])
                ld = plsc.load_gather(lab_v, [dv])
                isneg = ls != ld
                s2 = sv + jnp.where(isneg, jnp.int32(N), jnp.int32(0))
                sx0_v[sl] = 2 * s2
                sx1_v[sl] = 2 * s2 + 1
                s00_v[sl] = 2 * sv
                s01_v[sl] = 2 * sv + 1
                d2_v[g, pl.ds(16 * j, 16)] = dv + jnp.where(
                    isneg, jnp.int32(SEG), jnp.int32(0))
            return ()

        lax.fori_loop(0, NCP, cbody, (), unroll=False)
        plsc.subcore_barrier()

        def dbody(g, _):
            pltpu.async_copy(ones_v, dacc.at[d2_v.at[g]], sd, add=True)
            pltpu.async_copy(d2_v.at[g], dst2_h.at[pl.ds(obase + g * BP, BP)], sd2)
            return ()

        lax.fori_loop(0, NCP, dbody, (), unroll=False)

        pltpu.sync_copy(sx0_v, srcx_h.at[pl.ds(obase, EPT)])
        pltpu.sync_copy(sx1_v, srcx_h.at[pl.ds(EPAD + obase, EPT)])
        pltpu.sync_copy(s00_v, srcx0_h.at[pl.ds(obase, EPT)])
        pltpu.sync_copy(s01_v, srcx0_h.at[pl.ds(EPAD + obase, EPT)])
        pltpu.sync_copy(dst_v, dst0_h.at[pl.ds(obase, EPT)])

        @pl.when(r == 1)
        def _pads():
            pbase = q * EPP + 2 * EPT
            pltpu.sync_copy(pz_v, srcx_h.at[pl.ds(pbase, NPAD)])
            pltpu.sync_copy(pz_v, srcx_h.at[pl.ds(EPAD + pbase, NPAD)])
            pltpu.sync_copy(pz_v, srcx0_h.at[pl.ds(pbase, NPAD)])
            pltpu.sync_copy(pz_v, srcx0_h.at[pl.ds(EPAD + pbase, NPAD)])
            pltpu.sync_copy(pd2_v, dst2_h.at[pl.ds(pbase, NPAD)])
            pltpu.sync_copy(pd0_v, dst0_h.at[pl.ds(pbase, NPAD)])

        def ddrain(g, _):
            pltpu.make_async_copy(ones_v, dacc.at[d2_v.at[g]], sd).wait()
            pltpu.make_async_copy(d2_v.at[g], dst2_h.at[pl.ds(obase + g * BP, BP)],
                                  sd2).wait()
            return ()

        lax.fori_loop(0, NCP, ddrain, (), unroll=False)
        plsc.subcore_barrier()
        pltpu.sync_copy(dacc.at[pl.ds(s * DSL, DSL)],
                        degs_h.at[pl.ds(c * S2 + s * DSL, DSL)])

    return k(src, dst, labels, zeros_deg)


def _sc_seg_pass(table_v, srcx, dst2, zeros, S):
    RPT = S // NSUB
    HALF = 2
    CPI = 2 * HALF
    QN = 4
    NCQ = NCH // QN
    NQ = NCQ // CPI

    @functools.partial(
        pl.kernel,
        mesh=_MESH,
        compiler_params=pltpu.CompilerParams(use_tc_tiling_on_sc=False),
        out_type=jax.ShapeDtypeStruct((NC, S, 64), jnp.float32),
        scratch_types=(
            [pltpu.VMEM((NCQ, B), jnp.int32),
             pltpu.VMEM((NCQ, B), jnp.int32)]
            + [pltpu.VMEM((B, 64), jnp.float32) for _ in range(CPI)]
            + [pltpu.VMEM_SHARED((S, 64), jnp.float32)]
            + [pltpu.SemaphoreType.DMA for _ in range(2 * CPI)]
        ),
    )
    def k(table_h, srcx_h, dst_h, zeros_h, out_h, sidx, didx,
          r0, r1, r2, r3, acc, g0, g1, g2, g3, t0, t1, t2, t3):
        rows = [r0, r1, r2, r3]
        sg = [g0, g1, g2, g3]
        ss = [t0, t1, t2, t3]
        c = lax.axis_index("c")
        s = lax.axis_index("s")
        w = c * NSUB + s
        pltpu.sync_copy(zeros_h.at[pl.ds(s * RPT, RPT)], acc.at[pl.ds(s * RPT, RPT)])
        plsc.subcore_barrier()

        def gissue(ch, b):
            pltpu.async_copy(table_h.at[sidx.at[ch]], rows[b], sg[b])

        def gwait(ch, b):
            pltpu.make_async_copy(table_h.at[sidx.at[ch]], rows[b], sg[b]).wait()

        def sissue(ch, b):
            pltpu.async_copy(rows[b], acc.at[didx.at[ch]], ss[b], add=True)

        def swait(ch, b):
            pltpu.make_async_copy(rows[b], acc.at[didx.at[ch]], ss[b]).wait()

        def quarter(q, _):
            pltpu.sync_copy(srcx_h.at[w, pl.ds(q * NCQ, NCQ)], sidx)
            pltpu.sync_copy(dst_h.at[s, pl.ds(q * NCQ, NCQ)], didx)
            for b in range(HALF):
                gissue(b, b)

            def body(t, _):
                for b in range(HALF):
                    gissue(CPI * t + HALF + b, HALF + b)
                for b in range(HALF):
                    gwait(CPI * t + b, b)
                    sissue(CPI * t + b, b)
                for b in range(HALF):

                    @pl.when(t < NQ - 1)
                    def _():
                        swait(CPI * t + b, b)
                        gissue(CPI * (t + 1) + b, b)

                for b in range(HALF):
                    gwait(CPI * t + HALF + b, HALF + b)
                    sissue(CPI * t + HALF + b, HALF + b)
                for b in range(HALF):
                    swait(CPI * t + HALF + b, HALF + b)
                return ()

            lax.fori_loop(0, NQ, body, (), unroll=False)
            for b in range(HALF):
                swait(CPI * (NQ - 1) + b, b)
            return ()

        lax.fori_loop(0, QN, quarter, (), unroll=False)
        plsc.subcore_barrier()
        pltpu.sync_copy(acc.at[pl.ds(s * RPT, RPT)],
                        out_h.at[c, pl.ds(s * RPT, RPT)])

    return k(table_v, srcx, dst2, zeros)


RB = 1000


def _cat_halves(a2):
    return jnp.concatenate([a2[0], a2[1]], axis=-1)


def _tc_init(feat, degs, Wt, bt, C1s, a0c):
    def body(f_r, dg_r, wt_r, bt_r, c1_r, a0_r, g1_r, th_r, d2_r, dsq_r, dv_r, y0_r):
        f = f_r[...] + a0_r[0, :, 0:1] * 0.0
        pd = dg_r[0, 0] + dg_r[1, 0]
        nd = dg_r[0, 1] + dg_r[1, 1]
        pdc = jnp.maximum(pd, 1.0)
        ndc = jnp.maximum(nd, 1.0)
        adc = jnp.maximum(pd + nd, 1.0)
        dp = lax.rsqrt(pdc)
        dn = lax.rsqrt(ndc)
        g1_r[0] = f * dp
        g1_r[1] = f * dn
        th_r[...] = jnp.dot(f, wt_r[...], preferred_element_type=jnp.float32) + bt_r[...]
        d2_r[0] = dp * dp
        d2_r[1] = dn * dn
        dsq_r[0] = jnp.sqrt(pdc)
        dsq_r[1] = jnp.sqrt(ndc)
        dv_r[...] = lax.rsqrt(adc)
        y0_r[...] = jnp.dot(f, c1_r[...], preferred_element_type=jnp.float32)

    grid = (N // RB,)
    return pl.pallas_call(
        body,
        grid=grid,
        in_specs=[
            pl.BlockSpec((RB, D), lambda i: (i, 0)),
            pl.BlockSpec((NC, 2, RB, 1), lambda i: (0, 0, i, 0)),
            pl.BlockSpec((D, OUT), lambda i: (0, 0)),
            pl.BlockSpec((1, OUT), lambda i: (0, 0)),
            pl.BlockSpec((D, OUT), lambda i: (0, 0)),
            pl.BlockSpec((NC, RB, 64), lambda i: (0, i, 0)),
        ],
        out_specs=[
            pl.BlockSpec((2, RB, D), lambda i: (0, i, 0)),
            pl.BlockSpec((RB, OUT), lambda i: (i, 0)),
            pl.BlockSpec((2, RB, 1), lambda i: (0, i, 0)),
            pl.BlockSpec((2, RB, 1), lambda i: (0, i, 0)),
            pl.BlockSpec((RB, 1), lambda i: (i, 0)),
            pl.BlockSpec((RB, OUT), lambda i: (i, 0)),
        ],
        out_shape=[
            jax.ShapeDtypeStruct((2, N, D), jnp.float32),
            jax.ShapeDtypeStruct((N, OUT), jnp.float32),
            jax.ShapeDtypeStruct((2, N, 1), jnp.float32),
            jax.ShapeDtypeStruct((2, N, 1), jnp.float32),
            jax.ShapeDtypeStruct((N, 1), jnp.float32),
            jax.ShapeDtypeStruct((N, OUT), jnp.float32),
        ],
    )(feat, degs, Wt, bt, C1s, a0c)


def _tc_update(G, AGGc, D2, Dsq, Ck, Y):
    def body(g_r, a_r, d2_r, dsq_r, ck_r, y_r, gn_r, yn_r):
        agg = jnp.stack([_cat_halves(a_r[:, 0]), _cat_halves(a_r[:, 1])])
        g = g_r[...] - d2_r[...] * agg
        gn_r[...] = g
        f = g * dsq_r[...]
        yn_r[...] = (y_r[...]
                     + jnp.dot(f[0], ck_r[0], preferred_element_type=jnp.float32)
                     + jnp.dot(f[1], ck_r[1], preferred_element_type=jnp.float32))

    grid = (N // RB,)
    return pl.pallas_call(
        body,
        grid=grid,
        in_specs=[
            pl.BlockSpec((2, RB, D), lambda i: (0, i, 0)),
            pl.BlockSpec((NC, 2, RB, 64), lambda i: (0, 0, i, 0)),
            pl.BlockSpec((2, RB, 1), lambda i: (0, i, 0)),
            pl.BlockSpec((2, RB, 1), lambda i: (0, i, 0)),
            pl.BlockSpec((2, D, OUT), lambda i: (0, 0, 0)),
            pl.BlockSpec((RB, OUT), lambda i: (i, 0)),
        ],
        out_specs=[
            pl.BlockSpec((2, RB, D), lambda i: (0, i, 0)),
            pl.BlockSpec((RB, OUT), lambda i: (i, 0)),
        ],
        out_shape=[
            jax.ShapeDtypeStruct((2, N, D), jnp.float32),
            jax.ShapeDtypeStruct((N, OUT), jnp.float32),
        ],
    )(G, AGGc, D2, Dsq, Ck, Y)


def _tc_final(G4, AGG4c, D2, Dsq, C5, Y, bl1, feat, A0c, Dv, Wa, Wb, bl):
    def body(g_r, a_r, d2_r, dsq_r, c5_r, y_r, bl1_r, f_r, a0_r, dv_r,
             wa_r, wb_r, bl_r, hso_r, hspn_r):
        agg = jnp.stack([_cat_halves(a_r[:, 0]), _cat_halves(a_r[:, 1])])
        g = g_r[...] - d2_r[...] * agg
        f5 = g * dsq_r[...]
        y = (y_r[...]
             + jnp.dot(f5[0], c5_r[0], preferred_element_type=jnp.float32)
             + jnp.dot(f5[1], c5_r[1], preferred_element_type=jnp.float32)
             + bl1_r[...])
        hspn_r[...] = jnp.where(y >= 0, y, NEG_SLOPE * y)
        cvec = _cat_halves(a0_r[...]) * dv_r[...]
        z = (jnp.dot(f_r[...], wa_r[...], preferred_element_type=jnp.float32)
             + jnp.dot(cvec, wb_r[...], preferred_element_type=jnp.float32)
             + bl_r[...])
        hso_r[...] = jnp.where(z >= 0, z, NEG_SLOPE * z)

    grid = (N // RB,)
    return pl.pallas_call(
        body,
        grid=grid,
        in_specs=[
            pl.BlockSpec((2, RB, D), lambda i: (0, i, 0)),
            pl.BlockSpec((NC, 2, RB, 64), lambda i: (0, 0, i, 0)),
            pl.BlockSpec((2, RB, 1), lambda i: (0, i, 0)),
            pl.BlockSpec((2, RB, 1), lambda i: (0, i, 0)),
            pl.BlockSpec((2, D, OUT), lambda i: (0, 0, 0)),
            pl.BlockSpec((RB, OUT), lambda i: (i, 0)),
            pl.BlockSpec((1, OUT), lambda i: (0, 0)),
            pl.BlockSpec((RB, D), lambda i: (i, 0)),
            pl.BlockSpec((NC, RB, 64), lambda i: (0, i, 0)),
            pl.BlockSpec((RB, 1), lambda i: (i, 0)),
            pl.BlockSpec((D, OUT), lambda i: (0, 0)),
            pl.BlockSpec((D, OUT), lambda i: (0, 0)),
            pl.BlockSpec((1, OUT), lambda i: (0, 0)),
        ],
        out_specs=[
            pl.BlockSpec((RB, OUT), lambda i: (i, 0)),
            pl.BlockSpec((RB, OUT), lambda i: (i, 0)),
        ],
        out_shape=[
            jax.ShapeDtypeStruct((N, OUT), jnp.float32),
            jax.ShapeDtypeStruct((N, OUT), jnp.float32),
        ],
    )(G4, AGG4c, D2, Dsq, C5, Y, bl1, feat, A0c, Dv, Wa, Wb, bl)


def kernel(feat, edge_index, labels, W_transh, b_transh, W_lin, b_lin,
           W_lin1, b_lin1):
    src = edge_index[0]
    dst = edge_index[1]

    t0, t1, t2, t3 = THETAS
    a = [sum(t) for t in THETAS]
    b = [-(2 * i * a[i] + THETAS[i][1] + 2 * THETAS[i][2]) for i in range(4)]
    WL = [W_lin[D * i:D * (i + 1)] for i in range(4)]
    Wa = a[0] * WL[0] + a[1] * WL[1] + a[2] * WL[2] + a[3] * WL[3]
    Wb = b[0] * WL[0] + b[1] * WL[1] + b[2] * WL[2] + b[3] * WL[3]
    V = [W_lin1[D * i:D * (i + 1)] for i in range(4)]
    A = [t0[0] * V[0], t0[1] * V[0], t0[2] * V[0] + t1[0] * V[1],
         t1[1] * V[1], t1[2] * V[1]]
    Bm = [t2[0] * V[2], t2[1] * V[2], t2[2] * V[2] + t3[0] * V[3],
          t3[1] * V[3], t3[2] * V[3]]
    C = [jnp.stack([Ak, Bk]) for Ak, Bk in zip(A, Bm)]
    C1s = A[0] + Bm[0]

    zeros_deg = jnp.zeros((S2,), jnp.float32)
    zeros_n = jnp.zeros((SEG, 64), jnp.float32)
    zeros_2n = jnp.zeros((S2, 64), jnp.float32)

    srcx, srcx0, dst2, dst0, degs_flat = _sc_prep(src, dst, labels, zeros_deg)
    degs = degs_flat.reshape(NC, 2, SEG, 1)
    srcx = srcx.reshape(NC * NSUB, NCH, B)
    srcx0 = srcx0.reshape(NC * NSUB, NCH, B)
    dst2 = dst2.reshape(NSUB, NCH, B)
    dst0 = dst0.reshape(NSUB, NCH, B)

    feat_v = feat.reshape(2 * N, 64)
    agg0c = _sc_seg_pass(feat_v, srcx0, dst0, zeros_n, SEG)

    pass

    bt = b_transh.reshape(1, OUT)
    bl = b_lin.reshape(1, OUT)
    bl1 = b_lin1.reshape(1, OUT)
    G, transh, D2, Dsq, Dv, Y = _tc_init(feat, degs, W_transh, bt, C1s, agg0c)

    for k in range(1, 4):
        AGGc = _sc_seg_pass(G.reshape(4 * N, 64), srcx, dst2, zeros_2n, S2)
        AGGc = AGGc.reshape(NC, 2, SEG, 64)
        G, Y = _tc_update(G, AGGc, D2, Dsq, C[k], Y)
    AGG4c = _sc_seg_pass(G.reshape(4 * N, 64), srcx, dst2, zeros_2n, S2)
    AGG4c = AGG4c.reshape(NC, 2, SEG, 64)

    hs_o_out, hs_pn_out = _tc_final(G, AGG4c, D2, Dsq, C[4], Y, bl1,
                                    feat, agg0c, Dv, Wa, Wb, bl)
    return (hs_o_out, hs_pn_out, transh)
